# initial kernel scaffold (unmeasured)
import jax
import jax.numpy as jnp
from jax import lax
from jax.experimental import pallas as pl
from jax.experimental.pallas import tpu as pltpu

N_DEV = 4
N_LAYERS = 3
B = 512
D = 256
B_OUT = B // N_DEV


def kernel(x, Win0, Wout0, Win1, Wout1, Win2, Wout2):
    def body(x_ref, win0_ref, wout0_ref, win1_ref, wout1_ref,
             win2_ref, wout2_ref, out_ref, comm_ref, send_sems, recv_sems):
        my = lax.axis_index("i")

        barrier_sem = pltpu.get_barrier_semaphore()
        for k in range(1, N_DEV):
            pl.semaphore_signal(
                barrier_sem, inc=1,
                device_id=((my + k) % N_DEV,),
                device_id_type=pl.DeviceIdType.MESH,
            )
        pl.semaphore_wait(barrier_sem, N_DEV - 1)

        win_refs = [win0_ref, win1_ref, win2_ref]
        wout_refs = [wout0_ref, wout1_ref, wout2_ref]

        xb = x_ref[...].astype(jnp.bfloat16)
        acc = None
        for L in range(N_LAYERS):
            h = jnp.maximum(
                jax.lax.dot_general(
                    xb, win_refs[L][...].astype(jnp.bfloat16),
                    (((1,), (0,)), ((), ())),
                    preferred_element_type=jnp.float32,
                ),
                0.0,
            )
            partial = jax.lax.dot_general(
                h.astype(jnp.bfloat16), wout_refs[L][...].astype(jnp.bfloat16),
                (((1,), (0,)), ((), ())),
                preferred_element_type=jnp.float32,
            )
            comm_ref[L, 0] = partial.astype(jnp.bfloat16)

            rdmas = []
            for k in range(1, N_DEV):
                rdma = pltpu.make_async_remote_copy(
                    src_ref=comm_ref.at[L, 0],
                    dst_ref=comm_ref.at[L, k],
                    send_sem=send_sems.at[L, k - 1],
                    recv_sem=recv_sems.at[L, k - 1],
                    device_id=((my + k) % N_DEV,),
                    device_id_type=pl.DeviceIdType.MESH,
                )
                rdma.start()
                rdmas.append(rdma)
            for rdma in rdmas:
                rdma.wait()

            acc = partial
            for k in range(1, N_DEV):
                acc = acc + comm_ref[L, k][...].astype(jnp.float32)
            xb = acc.astype(jnp.bfloat16)

        out_ref[...] = lax.dynamic_slice_in_dim(acc, my * B_OUT, B_OUT, 0)

    return pl.pallas_call(
        body,
        out_shape=jax.ShapeDtypeStruct((B_OUT, D), jnp.float32),
        in_specs=[pl.BlockSpec(memory_space=pltpu.VMEM)] * 7,
        out_specs=pl.BlockSpec(memory_space=pltpu.VMEM),
        scratch_shapes=[
            pltpu.VMEM((N_LAYERS, N_DEV, B, D), jnp.bfloat16),
            pltpu.SemaphoreType.DMA((N_LAYERS, N_DEV - 1)),
            pltpu.SemaphoreType.DMA((N_LAYERS, N_DEV - 1)),
        ],
        compiler_params=pltpu.CompilerParams(collective_id=0),
    )(x, Win0, Wout0, Win1, Wout1, Win2, Wout2)


# baseline (device time: 36305 ns/iter reference)
import jax
import jax.numpy as jnp
from jax import lax
from jax.experimental import pallas as pl
from jax.experimental.pallas import tpu as pltpu

N_DEV = 4
N_LAYERS = 3
B = 512
D = 256
B_OUT = B // N_DEV


def kernel(x, Win0, Wout0, Win1, Wout1, Win2, Wout2):
    def body(x_ref, win0_ref, wout0_ref, win1_ref, wout1_ref,
             win2_ref, wout2_ref, out_ref, comm_ref, acc_ref,
             send_sems, recv_sems):
        my = lax.axis_index("i")

        barrier_sem = pltpu.get_barrier_semaphore()
        for k in range(1, N_DEV):
            pl.semaphore_signal(
                barrier_sem, inc=1,
                device_id=((my + k) % N_DEV,),
                device_id_type=pl.DeviceIdType.MESH,
            )
        pl.semaphore_wait(barrier_sem, N_DEV - 1)

        win_refs = [win0_ref, win1_ref, win2_ref]
        wout_refs = [wout0_ref, wout1_ref, wout2_ref]

        xb = x_ref[...].astype(jnp.bfloat16)
        acc = None
        for L in range(N_LAYERS):
            h = jnp.maximum(
                jax.lax.dot_general(
                    xb, win_refs[L][...].astype(jnp.bfloat16),
                    (((1,), (0,)), ((), ())),
                    preferred_element_type=jnp.float32,
                ),
                0.0,
            )
            partial = jax.lax.dot_general(
                h.astype(jnp.bfloat16), wout_refs[L][...].astype(jnp.bfloat16),
                (((1,), (0,)), ((), ())),
                preferred_element_type=jnp.float32,
            )
            comm_ref[L, 0] = partial.astype(jnp.bfloat16)

            rdmas = []
            for k in range(1, N_DEV):
                rdma = pltpu.make_async_remote_copy(
                    src_ref=comm_ref.at[L, 0],
                    dst_ref=comm_ref.at[L, k],
                    send_sem=send_sems.at[L, k - 1],
                    recv_sem=recv_sems.at[L, k - 1],
                    device_id=((my + k) % N_DEV,),
                    device_id_type=pl.DeviceIdType.MESH,
                )
                rdma.start()
                rdmas.append(rdma)
            for rdma in rdmas:
                rdma.wait()

            acc = partial
            for k in range(1, N_DEV):
                acc = acc + comm_ref[L, k][...].astype(jnp.float32)
            xb = acc.astype(jnp.bfloat16)

        acc_ref[...] = acc
        out_ref[...] = acc_ref[pl.ds(my * B_OUT, B_OUT), :]

    return pl.pallas_call(
        body,
        out_shape=jax.ShapeDtypeStruct((B_OUT, D), jnp.float32),
        in_specs=[pl.BlockSpec(memory_space=pltpu.VMEM)] * 7,
        out_specs=pl.BlockSpec(memory_space=pltpu.VMEM),
        scratch_shapes=[
            pltpu.VMEM((N_LAYERS, N_DEV, B, D), jnp.bfloat16),
            pltpu.VMEM((B, D), jnp.float32),
            pltpu.SemaphoreType.DMA((N_LAYERS, N_DEV - 1)),
            pltpu.SemaphoreType.DMA((N_LAYERS, N_DEV - 1)),
        ],
        compiler_params=pltpu.CompilerParams(collective_id=0),
    )(x, Win0, Wout0, Win1, Wout1, Win2, Wout2)


# device time: 30731 ns/iter; 1.1814x vs baseline; 1.1814x over previous
import jax
import jax.numpy as jnp
from jax import lax
from jax.experimental import pallas as pl
from jax.experimental.pallas import tpu as pltpu

N_DEV = 4
N_LAYERS = 3
B = 512
D = 256
B_OUT = B // N_DEV


def kernel(x, Win0, Wout0, Win1, Wout1, Win2, Wout2):
    def body(x_ref, win0_ref, wout0_ref, win1_ref, wout1_ref,
             win2_ref, wout2_ref, out_ref,
             part_ref, rs_ref, red_ref, ag_ref, xbuf_ref,
             rs_send, rs_recv, ag_send, ag_recv):
        my = lax.axis_index("i")

        barrier_sem = pltpu.get_barrier_semaphore()
        for k in range(1, N_DEV):
            pl.semaphore_signal(
                barrier_sem, inc=1,
                device_id=((my + k) % N_DEV,),
                device_id_type=pl.DeviceIdType.MESH,
            )
        pl.semaphore_wait(barrier_sem, N_DEV - 1)

        win_refs = [win0_ref, win1_ref, win2_ref]
        wout_refs = [wout0_ref, wout1_ref, wout2_ref]

        xb = x_ref[...].astype(jnp.bfloat16)
        for L in range(N_LAYERS):
            h = jnp.maximum(
                jax.lax.dot_general(
                    xb, win_refs[L][...].astype(jnp.bfloat16),
                    (((1,), (0,)), ((), ())),
                    preferred_element_type=jnp.float32,
                ),
                0.0,
            )
            partial = jax.lax.dot_general(
                h.astype(jnp.bfloat16), wout_refs[L][...].astype(jnp.bfloat16),
                (((1,), (0,)), ((), ())),
                preferred_element_type=jnp.float32,
            )
            part_ref[L] = partial.astype(jnp.bfloat16)

            rs_rdmas = []
            for k in range(1, N_DEV):
                tgt = (my + k) % N_DEV
                rdma = pltpu.make_async_remote_copy(
                    src_ref=part_ref.at[L, pl.ds(tgt * B_OUT, B_OUT), :],
                    dst_ref=rs_ref.at[L, k - 1],
                    send_sem=rs_send.at[L, k - 1],
                    recv_sem=rs_recv.at[L, k - 1],
                    device_id=(tgt,),
                    device_id_type=pl.DeviceIdType.MESH,
                )
                rdma.start()
                rs_rdmas.append(rdma)
            for rdma in rs_rdmas:
                rdma.wait()

            red = part_ref[L, pl.ds(my * B_OUT, B_OUT), :].astype(jnp.float32)
            for k in range(1, N_DEV):
                red = red + rs_ref[L, k - 1].astype(jnp.float32)

            if L == N_LAYERS - 1:
                out_ref[...] = red
            else:
                red_ref[L] = red.astype(jnp.bfloat16)
                ag_rdmas = []
                for k in range(1, N_DEV):
                    rdma = pltpu.make_async_remote_copy(
                        src_ref=red_ref.at[L],
                        dst_ref=ag_ref.at[L, k - 1],
                        send_sem=ag_send.at[L, k - 1],
                        recv_sem=ag_recv.at[L, k - 1],
                        device_id=((my + k) % N_DEV,),
                        device_id_type=pl.DeviceIdType.MESH,
                    )
                    rdma.start()
                    ag_rdmas.append(rdma)
                xbuf_ref[pl.ds(my * B_OUT, B_OUT), :] = red_ref[L]
                for k, rdma in zip(range(1, N_DEV), ag_rdmas):
                    rdma.wait()
                    src_pos = (my - k) % N_DEV
                    xbuf_ref[pl.ds(src_pos * B_OUT, B_OUT), :] = ag_ref[L, k - 1]
                xb = xbuf_ref[...]

    return pl.pallas_call(
        body,
        out_shape=jax.ShapeDtypeStruct((B_OUT, D), jnp.float32),
        in_specs=[pl.BlockSpec(memory_space=pltpu.VMEM)] * 7,
        out_specs=pl.BlockSpec(memory_space=pltpu.VMEM),
        scratch_shapes=[
            pltpu.VMEM((N_LAYERS, B, D), jnp.bfloat16),
            pltpu.VMEM((N_LAYERS, N_DEV - 1, B_OUT, D), jnp.bfloat16),
            pltpu.VMEM((N_LAYERS - 1, B_OUT, D), jnp.bfloat16),
            pltpu.VMEM((N_LAYERS - 1, N_DEV - 1, B_OUT, D), jnp.bfloat16),
            pltpu.VMEM((B, D), jnp.bfloat16),
            pltpu.SemaphoreType.DMA((N_LAYERS, N_DEV - 1)),
            pltpu.SemaphoreType.DMA((N_LAYERS, N_DEV - 1)),
            pltpu.SemaphoreType.DMA((N_LAYERS - 1, N_DEV - 1)),
            pltpu.SemaphoreType.DMA((N_LAYERS - 1, N_DEV - 1)),
        ],
        compiler_params=pltpu.CompilerParams(collective_id=0),
    )(x, Win0, Wout0, Win1, Wout1, Win2, Wout2)


# device time: 9649 ns/iter; 3.7626x vs baseline; 3.1849x over previous
import jax
import jax.numpy as jnp
from jax import lax
from jax.experimental import pallas as pl
from jax.experimental.pallas import tpu as pltpu

N_DEV = 4
N_LAYERS = 3
B = 512
D = 256
B_OUT = B // N_DEV


def kernel(x, Win0, Wout0, Win1, Wout1, Win2, Wout2):
    def body(x_ref, win0_ref, wout0_ref, win1_ref, wout1_ref,
             win2_ref, wout2_ref, out_ref,
             part_ref, rs_ref, red_ref, ag_ref,
             rs_send, rs_recv, ag_send, ag_recv):
        my = lax.axis_index("i")

        barrier_sem = pltpu.get_barrier_semaphore()
        for k in range(1, N_DEV):
            pl.semaphore_signal(
                barrier_sem, inc=1,
                device_id=((my + k) % N_DEV,),
                device_id_type=pl.DeviceIdType.MESH,
            )
        pl.semaphore_wait(barrier_sem, N_DEV - 1)

        win_refs = [win0_ref, win1_ref, win2_ref]
        wout_refs = [wout0_ref, wout1_ref, wout2_ref]
        all_descs = []

        def block_out(xj, wi, wo):
            h = jnp.maximum(
                jax.lax.dot_general(
                    xj, wi, (((1,), (0,)), ((), ())),
                    preferred_element_type=jnp.float32,
                ),
                0.0,
            )
            return jax.lax.dot_general(
                h.astype(jnp.bfloat16), wo, (((1,), (0,)), ((), ())),
                preferred_element_type=jnp.float32,
            )

        def rs_send_block(L, idx, tgt, partial_j):
            part_ref[L, idx] = partial_j.astype(jnp.bfloat16)
            d = pltpu.make_async_remote_copy(
                src_ref=part_ref.at[L, idx],
                dst_ref=rs_ref.at[L, idx],
                send_sem=rs_send.at[L, idx],
                recv_sem=rs_recv.at[L, idx],
                device_id=(tgt,),
                device_id_type=pl.DeviceIdType.MESH,
            )
            d.start()
            all_descs.append(d)
            return d

        def reduce_blocks(L, partial_own, rs_descs):
            for d in rs_descs:
                d.wait_recv()
            red = partial_own
            for i in range(N_DEV - 1):
                red = red + rs_ref[L, i].astype(jnp.float32)
            return red

        def ag_start(L, red):
            red_ref[L] = red.astype(jnp.bfloat16)
            descs = {}
            for k in range(1, N_DEV):
                d = pltpu.make_async_remote_copy(
                    src_ref=red_ref.at[L],
                    dst_ref=ag_ref.at[L, k - 1],
                    send_sem=ag_send.at[L, k - 1],
                    recv_sem=ag_recv.at[L, k - 1],
                    device_id=((my + k) % N_DEV,),
                    device_id_type=pl.DeviceIdType.MESH,
                )
                d.start()
                descs[k] = d
                all_descs.append(d)
            return descs

        wi = win_refs[0][...].astype(jnp.bfloat16)
        wo = wout_refs[0][...].astype(jnp.bfloat16)
        rs_descs = []
        for k in (2, 1, 3):
            tgt = (my + k) % N_DEV
            xj = x_ref[pl.ds(tgt * B_OUT, B_OUT), :].astype(jnp.bfloat16)
            rs_descs.append(rs_send_block(0, k - 1, tgt, block_out(xj, wi, wo)))
        xown = x_ref[pl.ds(my * B_OUT, B_OUT), :].astype(jnp.bfloat16)
        partial_own = block_out(xown, wi, wo)
        red = reduce_blocks(0, partial_own, rs_descs)
        ag_descs = ag_start(0, red)

        for L in (1, 2):
            wi = win_refs[L][...].astype(jnp.bfloat16)
            wo = wout_refs[L][...].astype(jnp.bfloat16)
            partial_own = block_out(red.astype(jnp.bfloat16), wi, wo)
            rs_descs = []
            for k in (1, 3, 2):
                ag_descs[k].wait_recv()
                tgt = (my - k) % N_DEV
                idx = 3 - k
                xj = ag_ref[L - 1, k - 1].astype(jnp.bfloat16)
                rs_descs.append(rs_send_block(L, idx, tgt, block_out(xj, wi, wo)))
            red = reduce_blocks(L, partial_own, rs_descs)
            if L < N_LAYERS - 1:
                ag_descs = ag_start(L, red)

        out_ref[...] = red

        for d in all_descs:
            d.wait_send()

    return pl.pallas_call(
        body,
        out_shape=jax.ShapeDtypeStruct((B_OUT, D), jnp.float32),
        in_specs=[pl.BlockSpec(memory_space=pltpu.VMEM)] * 7,
        out_specs=pl.BlockSpec(memory_space=pltpu.VMEM),
        scratch_shapes=[
            pltpu.VMEM((N_LAYERS, N_DEV - 1, B_OUT, D), jnp.bfloat16),
            pltpu.VMEM((N_LAYERS, N_DEV - 1, B_OUT, D), jnp.bfloat16),
            pltpu.VMEM((N_LAYERS - 1, B_OUT, D), jnp.bfloat16),
            pltpu.VMEM((N_LAYERS - 1, N_DEV - 1, B_OUT, D), jnp.bfloat16),
            pltpu.SemaphoreType.DMA((N_LAYERS, N_DEV - 1)),
            pltpu.SemaphoreType.DMA((N_LAYERS, N_DEV - 1)),
            pltpu.SemaphoreType.DMA((N_LAYERS - 1, N_DEV - 1)),
            pltpu.SemaphoreType.DMA((N_LAYERS - 1, N_DEV - 1)),
        ],
        compiler_params=pltpu.CompilerParams(collective_id=0),
    )(x, Win0, Wout0, Win1, Wout1, Win2, Wout2)
